# Initial kernel scaffold; baseline (speedup 1.0000x reference)
import jax
import jax.numpy as jnp
from jax.experimental import pallas as pl


def kernel(atomic_energy, atom_types, shifts, scales):
    raise NotImplementedError("write your pallas kernel here")


# speedup vs baseline: 3.6034x; 3.6034x over previous
import functools

import jax
import jax.numpy as jnp
from jax import lax
from jax.experimental import pallas as pl
from jax.experimental.pallas import tpu as pltpu
from jax.experimental.pallas import tpu_sc as plsc

N_ATOMS = 100000
NUM_TYPES = 64
L = 16
NUM_WORKERS = 32

CHUNK = 3136
N_PAD = NUM_WORKERS * CHUNK


def _sc_body(energy_hbm, types_hbm, shifts_hbm, scales_hbm, out_hbm,
             shifts_v, scales_v, types_v, energy_v, out_v):
    wid = lax.axis_index("s") * 2 + lax.axis_index("c")
    base = wid * CHUNK

    pltpu.sync_copy(shifts_hbm, shifts_v)
    pltpu.sync_copy(scales_hbm, scales_v)
    pltpu.sync_copy(types_hbm.at[pl.ds(base, CHUNK)], types_v)
    pltpu.sync_copy(energy_hbm.at[pl.ds(base, CHUNK)], energy_v)

    def step(i, carry):
        off = i * L
        t = types_v[pl.ds(off, L)]
        sh = plsc.load_gather(shifts_v, [t])
        sc = plsc.load_gather(scales_v, [t])
        e = energy_v[pl.ds(off, L)]
        out_v[pl.ds(off, L)] = sh + sc * e
        return carry

    lax.fori_loop(0, CHUNK // L, step, 0)

    pltpu.sync_copy(out_v, out_hbm.at[pl.ds(base, CHUNK)])


@jax.jit
def _run(energy_pad, types_pad, shifts, scales):
    mesh = plsc.VectorSubcoreMesh(core_axis_name="c", subcore_axis_name="s")
    return pl.kernel(
        _sc_body,
        out_type=jax.ShapeDtypeStruct((N_PAD,), jnp.float32),
        mesh=mesh,
        compiler_params=pltpu.CompilerParams(needs_layout_passes=False),
        scratch_types=[
            pltpu.VMEM((NUM_TYPES,), jnp.float32),
            pltpu.VMEM((NUM_TYPES,), jnp.float32),
            pltpu.VMEM((CHUNK,), jnp.int32),
            pltpu.VMEM((CHUNK,), jnp.float32),
            pltpu.VMEM((CHUNK,), jnp.float32),
        ],
    )(energy_pad, types_pad, shifts, scales)


def kernel(atomic_energy, atom_types, shifts, scales):
    n = atomic_energy.shape[0]
    energy = atomic_energy.astype(jnp.float32).reshape(-1)
    types = atom_types.astype(jnp.int32).reshape(-1)
    energy_pad = jnp.pad(energy, (0, N_PAD - n))
    types_pad = jnp.pad(types, (0, N_PAD - n))
    out = _run(energy_pad, types_pad, shifts, scales)
    return out[:n].reshape(-1, 1)


# speedup vs baseline: 3.9124x; 1.0857x over previous
import functools

import jax
import jax.numpy as jnp
from jax import lax
from jax.experimental import pallas as pl
from jax.experimental.pallas import tpu as pltpu
from jax.experimental.pallas import tpu_sc as plsc

N_ATOMS = 100000
NUM_TYPES = 64
L = 16
NUM_WORKERS = 32

CHUNK = 3136
N_PAD = NUM_WORKERS * CHUNK


def _sc_body(energy_hbm, types_hbm, shifts_hbm, scales_hbm, out_hbm,
             shifts_v, scales_v, types_v, energy_v, out_v, sem):
    wid = lax.axis_index("s") * 2 + lax.axis_index("c")
    base = wid * CHUNK

    c1 = pltpu.async_copy(shifts_hbm, shifts_v, sem)
    c2 = pltpu.async_copy(scales_hbm, scales_v, sem)
    c3 = pltpu.async_copy(types_hbm.at[pl.ds(base, CHUNK)], types_v, sem)
    c4 = pltpu.async_copy(energy_hbm.at[pl.ds(base, CHUNK)], energy_v, sem)
    c1.wait()
    c2.wait()
    c3.wait()
    c4.wait()

    @plsc.parallel_loop(0, CHUNK, L, unroll=8)
    def _(off):
        t = types_v[pl.ds(off, L)]
        sh = plsc.load_gather(shifts_v, [t])
        sc = plsc.load_gather(scales_v, [t])
        e = energy_v[pl.ds(off, L)]
        out_v[pl.ds(off, L)] = sh + sc * e

    pltpu.sync_copy(out_v, out_hbm.at[pl.ds(base, CHUNK)])


@jax.jit
def _run(energy_pad, types_pad, shifts, scales):
    mesh = plsc.VectorSubcoreMesh(core_axis_name="c", subcore_axis_name="s")
    return pl.kernel(
        _sc_body,
        out_type=jax.ShapeDtypeStruct((N_PAD,), jnp.float32),
        mesh=mesh,
        compiler_params=pltpu.CompilerParams(needs_layout_passes=False),
        scratch_types=[
            pltpu.VMEM((NUM_TYPES,), jnp.float32),
            pltpu.VMEM((NUM_TYPES,), jnp.float32),
            pltpu.VMEM((CHUNK,), jnp.int32),
            pltpu.VMEM((CHUNK,), jnp.float32),
            pltpu.VMEM((CHUNK,), jnp.float32),
            pltpu.SemaphoreType.DMA,
        ],
    )(energy_pad, types_pad, shifts, scales)


def kernel(atomic_energy, atom_types, shifts, scales):
    n = atomic_energy.shape[0]
    energy = atomic_energy.astype(jnp.float32).reshape(-1)
    types = atom_types.astype(jnp.int32).reshape(-1)
    energy_pad = jnp.pad(energy, (0, N_PAD - n))
    types_pad = jnp.pad(types, (0, N_PAD - n))
    out = _run(energy_pad, types_pad, shifts, scales)
    return out[:n].reshape(-1, 1)


# speedup vs baseline: 4.1127x; 1.0512x over previous
import jax
import jax.numpy as jnp
from jax import lax
from jax.experimental import pallas as pl
from jax.experimental.pallas import tpu as pltpu
from jax.experimental.pallas import tpu_sc as plsc

N_ATOMS = 100000
NUM_TYPES = 64
L = 16
NUM_WORKERS = 32

CHUNK = 3136
LAST = N_ATOMS - (NUM_WORKERS - 1) * CHUNK


def _sc_body(energy_hbm, types_hbm, shifts_hbm, scales_hbm, out_hbm,
             shifts_v, scales_v, types_v, energy_v, out_v, sem_tab, sem):
    wid = lax.axis_index("s") * 2 + lax.axis_index("c")
    base = wid * CHUNK

    t1 = pltpu.async_copy(shifts_hbm, shifts_v, sem_tab)
    t2 = pltpu.async_copy(scales_hbm, scales_v, sem_tab)

    def work(size):
        c1 = pltpu.async_copy(types_hbm.at[pl.ds(base, size)],
                              types_v.at[pl.ds(0, size)], sem)
        c2 = pltpu.async_copy(energy_hbm.at[pl.ds(base, size)],
                              energy_v.at[pl.ds(0, size)], sem)
        t1.wait()
        t2.wait()
        c1.wait()
        c2.wait()

        @plsc.parallel_loop(0, size, L, unroll=8)
        def _(off):
            t = types_v[pl.ds(off, L)]
            sh = plsc.load_gather(shifts_v, [t])
            sc = plsc.load_gather(scales_v, [t])
            e = energy_v[pl.ds(off, L)]
            out_v[pl.ds(off, L)] = sh + sc * e

        pltpu.sync_copy(out_v.at[pl.ds(0, size)],
                        out_hbm.at[pl.ds(base, size)])

    @pl.when(wid < NUM_WORKERS - 1)
    def _():
        work(CHUNK)

    @pl.when(wid == NUM_WORKERS - 1)
    def _():
        work(LAST)


@jax.jit
def _run(energy, types, shifts, scales):
    mesh = plsc.VectorSubcoreMesh(core_axis_name="c", subcore_axis_name="s")
    return pl.kernel(
        _sc_body,
        out_type=jax.ShapeDtypeStruct((N_ATOMS,), jnp.float32),
        mesh=mesh,
        compiler_params=pltpu.CompilerParams(needs_layout_passes=False),
        scratch_types=[
            pltpu.VMEM((NUM_TYPES,), jnp.float32),
            pltpu.VMEM((NUM_TYPES,), jnp.float32),
            pltpu.VMEM((CHUNK,), jnp.int32),
            pltpu.VMEM((CHUNK,), jnp.float32),
            pltpu.VMEM((CHUNK,), jnp.float32),
            pltpu.SemaphoreType.DMA,
            pltpu.SemaphoreType.DMA,
        ],
    )(energy, types, shifts, scales)


def kernel(atomic_energy, atom_types, shifts, scales):
    energy = atomic_energy.astype(jnp.float32).reshape(-1)
    types = atom_types.astype(jnp.int32).reshape(-1)
    out = _run(energy, types, shifts, scales)
    return out.reshape(-1, 1)
